# baseline (speedup 1.0000x reference)
import functools

import jax
import jax.numpy as jnp
from jax import lax
from jax.experimental import pallas as pl
from jax.experimental.pallas import tpu as pltpu
from jax.experimental.pallas import tpu_sc as plsc

BATCH = 16384
EMBED = 64

_info = plsc.get_sparse_core_info()
_NC, _NS, _L = _info.num_cores, _info.num_subcores, _info.num_lanes
_NW = _NC * _NS
_BPW = BATCH // _NW
_CHUNK = 128
_NCHUNK = _BPW // _CHUNK


def _rsqrt_nr(sv):
    bits = lax.bitcast_convert_type(sv, jnp.int32)
    y = lax.bitcast_convert_type(jnp.int32(0x5F3759DF) - (bits >> 1),
                                 jnp.float32)
    for _ in range(3):
        y = y * (1.5 - 0.5 * sv * y * y)
    return y


def _lane_sum(x):
    lanes = lax.iota(jnp.int32, _L)
    dnums = lax.GatherDimensionNumbers(
        offset_dims=(), collapsed_slice_dims=(0,), start_index_map=(0,))
    for sh in (8, 4, 2, 1):
        x = x + lax.gather(
            x, (lanes ^ sh)[:, None], dnums, slice_sizes=(1,),
            mode=lax.GatherScatterMode.PROMISE_IN_BOUNDS)
    return x


def _normalize_rows(rows_ref):
    def body(i, carry):
        chunks = []
        ss = jnp.zeros((_L,), jnp.float32)
        for c in range(EMBED // _L):
            r = rows_ref[i, pl.ds(c * _L, _L)]
            chunks.append(r)
            ss = ss + r * r
        y = _rsqrt_nr(jnp.maximum(_lane_sum(ss), jnp.float32(1e-24)))
        for c in range(EMBED // _L):
            rows_ref[i, pl.ds(c * _L, _L)] = chunks[c] * y
        return carry

    lax.fori_loop(0, _BPW, body, 0)


_mesh = plsc.VectorSubcoreMesh(core_axis_name="c", subcore_axis_name="s")


@functools.partial(
    pl.kernel,
    out_type=(jax.ShapeDtypeStruct((BATCH, EMBED), jnp.float32),
              jax.ShapeDtypeStruct((BATCH, EMBED), jnp.float32)),
    mesh=_mesh,
    compiler_params=pltpu.CompilerParams(use_tc_tiling_on_sc=False),
    scratch_types=[
        pltpu.VMEM((_NCHUNK, _CHUNK), jnp.int32),
        pltpu.VMEM((_NCHUNK, _CHUNK), jnp.int32),
        pltpu.VMEM((_BPW, EMBED), jnp.float32),
        pltpu.VMEM((_BPW, EMBED), jnp.float32),
        pltpu.SemaphoreType.DMA,
        pltpu.SemaphoreType.DMA,
    ],
)
def _emb_kernel(uids_hbm, iids_hbm, utab_hbm, itab_hbm,
                uout_hbm, iout_hbm,
                uidx, iidx, urows, irows, usem, isem):
    wid = lax.axis_index("s") * _NC + lax.axis_index("c")
    base = wid * _BPW

    for j in range(_NCHUNK):
        pltpu.sync_copy(uids_hbm.at[pl.ds(base + j * _CHUNK, _CHUNK)],
                        uidx.at[j])
        pltpu.sync_copy(iids_hbm.at[pl.ds(base + j * _CHUNK, _CHUNK)],
                        iidx.at[j])

    ucopies = []
    icopies = []
    for j in range(_NCHUNK):
        ucopies.append(pltpu.async_copy(
            utab_hbm.at[uidx.at[j]],
            urows.at[pl.ds(j * _CHUNK, _CHUNK)], usem))
    for j in range(_NCHUNK):
        icopies.append(pltpu.async_copy(
            itab_hbm.at[iidx.at[j]],
            irows.at[pl.ds(j * _CHUNK, _CHUNK)], isem))

    for cp in ucopies:
        cp.wait()
    _normalize_rows(urows)
    pltpu.sync_copy(urows, uout_hbm.at[pl.ds(base, _BPW)])

    for cp in icopies:
        cp.wait()
    _normalize_rows(irows)
    pltpu.sync_copy(irows, iout_hbm.at[pl.ds(base, _BPW)])


def kernel(user_ids, user_features, item_ids, item_features,
           user_table, item_table):
    del user_features, item_features
    uids = user_ids.astype(jnp.int32)
    iids = item_ids.astype(jnp.int32)
    return _emb_kernel(uids, iids, user_table, item_table)


# speedup vs baseline: 2.6349x; 2.6349x over previous
import functools

import jax
import jax.numpy as jnp
from jax import lax
from jax.experimental import pallas as pl
from jax.experimental.pallas import tpu as pltpu
from jax.experimental.pallas import tpu_sc as plsc

BATCH = 16384
EMBED = 64
NROWS = 1000000

_info = plsc.get_sparse_core_info()
_NC, _NS, _L = _info.num_cores, _info.num_subcores, _info.num_lanes
_NW = _NC * _NS
_WIN = 128
_NWIN = NROWS // _WIN
_WPT = _NWIN // _NW
_TAIL0 = _NWIN * _WIN
_CAP = 2048
_LCAP = _CAP + 16


def _rsqrt_nr(sv):
    bits = lax.bitcast_convert_type(sv, jnp.int32)
    y = lax.bitcast_convert_type(jnp.int32(0x5F3759DF) - (bits >> 1),
                                 jnp.float32)
    for _ in range(3):
        y = y * (1.5 - 0.5 * sv * y * y)
    return y


def _shuffle(x, idx):
    dnums = lax.GatherDimensionNumbers(
        offset_dims=(), collapsed_slice_dims=(0,), start_index_map=(0,))
    return lax.gather(x, idx[:, None], dnums, slice_sizes=(1,),
                      mode=lax.GatherScatterMode.PROMISE_IN_BOUNDS)


def _lane_sum(x):
    lanes = lax.iota(jnp.int32, _L)
    for sh in (8, 4, 2, 1):
        x = x + _shuffle(x, lanes ^ sh)
    return x


def _prefix_sum(x):
    lanes = lax.iota(jnp.int32, _L)
    for k, sh in enumerate((1, 2, 4, 8)):
        g = jnp.minimum(lanes, sh) >> k
        x = x + _shuffle(x, jnp.maximum(lanes - sh, 0)) * g
    return x


_mesh = plsc.VectorSubcoreMesh(core_axis_name="c", subcore_axis_name="s")


@functools.partial(
    pl.kernel,
    out_type=(jax.ShapeDtypeStruct((BATCH * EMBED,), jnp.float32),
              jax.ShapeDtypeStruct((BATCH * EMBED,), jnp.float32)),
    mesh=_mesh,
    compiler_params=pltpu.CompilerParams(needs_layout_passes=False),
    scratch_types=[
        pltpu.VMEM((BATCH,), jnp.int32),
        pltpu.VMEM((BATCH,), jnp.int32),
        pltpu.VMEM((_LCAP,), jnp.int32),
        pltpu.VMEM((_LCAP,), jnp.int32),
        pltpu.VMEM((_LCAP,), jnp.int32),
        pltpu.VMEM((_LCAP,), jnp.int32),
        pltpu.VMEM((_LCAP,), jnp.int32),
        pltpu.VMEM((_LCAP,), jnp.int32),
        pltpu.VMEM((_LCAP,), jnp.int32),
        pltpu.VMEM((_LCAP,), jnp.int32),
        pltpu.SMEM((272,), jnp.int32),
        pltpu.SMEM((272,), jnp.int32),
        pltpu.SMEM((272,), jnp.int32),
        pltpu.SMEM((272,), jnp.int32),
        pltpu.VMEM((2, EMBED, _WIN), jnp.float32),
        pltpu.VMEM((2, EMBED, _WIN), jnp.float32),
        pltpu.VMEM((EMBED, EMBED), jnp.float32),
        pltpu.VMEM((128, EMBED), jnp.float32),
        pltpu.VMEM((128, EMBED), jnp.float32),
        pltpu.VMEM((EMBED,), jnp.float32),
        pltpu.SemaphoreType.DMA,
        pltpu.SemaphoreType.DMA,
        pltpu.SemaphoreType.DMA,
        pltpu.SemaphoreType.DMA,
        pltpu.SemaphoreType.DMA,
        pltpu.SemaphoreType.DMA,
    ],
)
def _emb_kernel(uids_hbm, iids_hbm, utt_hbm, itt_hbm,
                uout_hbm, iout_hbm,
                uall, iall, usel_id, usel_pos, isel_id, isel_pos,
                uord_id, uord_pos, iord_id, iord_pos,
                hist, ustarts, istarts, cursor,
                ubuf, ibuf, tailbuf, uring, iring, drain_dst,
                usem0, usem1, isem0, isem1, uwsem, iwsem):
    wid = lax.axis_index("s") * _NC + lax.axis_index("c")
    wsems = (usem0, usem1, isem0, isem1)

    pltpu.sync_copy(uids_hbm, uall)
    pltpu.sync_copy(iids_hbm, iall)

    def sload(ref, j):
        return ref[pl.ds(j, _L)][0]

    def sstore(ref, j, val):
        plsc.store_scatter(ref, [jnp.full((_L,), j, jnp.int32)],
                           jnp.full((_L,), val, jnp.int32))

    def run_table(all_ref, sel_id, sel_pos, ord_id, ord_pos, starts,
                  tt_hbm, out_hbm, buf, ring, sem_a, sem_b, wsem):
        trash = jnp.full((_L,), _CAP + 8, jnp.int32)
        lanes15 = jnp.full((_L,), _L - 1, jnp.int32)
        ones = jnp.ones((_L,), jnp.int32)

        def sbody(v, cnt_vec):
            vec = all_ref[pl.ds(v * _L, _L)]
            diff = ((vec >> 7) & (_NW - 1)) ^ wid
            mi = ones - jnp.minimum(diff, ones)
            posv = lax.iota(jnp.int32, _L) + v * _L
            inc = _prefix_sum(mi)
            dest = (cnt_vec + inc - mi) * mi + trash * (ones - mi)
            plsc.store_scatter(sel_id, [dest], vec)
            plsc.store_scatter(sel_pos, [dest], posv)
            return jnp.minimum(cnt_vec + _shuffle(inc, lanes15), _CAP)
        cnt_vec = lax.fori_loop(0, BATCH // _L, sbody,
                                jnp.zeros((_L,), jnp.int32))
        cnt = cnt_vec[0]

        def zbody(l, c):
            hist[l] = 0
            return c
        lax.fori_loop(0, 246, zbody, 0)

        def hbody(j, c):
            lw = sload(sel_id, j) >> 12
            hist[lw] = hist[lw] + 1
            return c
        lax.fori_loop(0, cnt, hbody, 0)

        def pbody(l, s):
            starts[l] = s
            cursor[l] = s
            return s + hist[l]
        total = lax.fori_loop(0, 246, pbody, 0)
        del total

        def scbody(j, c):
            idv = sload(sel_id, j)
            lw = idv >> 12
            o = cursor[lw]
            cursor[lw] = o + 1
            sstore(ord_id, o, idv)
            sstore(ord_pos, o, sload(sel_pos, j))
            return c
        lax.fori_loop(0, cnt, scbody, 0)

        def start_win(l, b, sem):
            win = l * _NW + wid
            pltpu.make_async_copy(
                tt_hbm.at[:, pl.ds(win * _WIN, _WIN)], buf.at[b], sem
            ).start()

        def wait_win(b, sem):
            pltpu.make_async_copy(
                tt_hbm.at[:, pl.ds(0, _WIN)], buf.at[b], sem
            ).wait()

        def process_id(j, wref):
            @pl.when(j >= 128)
            def _():
                pltpu.make_async_copy(
                    out_hbm.at[pl.ds(0, EMBED)], drain_dst, wsem
                ).wait()
            idv = sload(ord_id, j)
            pos = sload(ord_pos, j)
            col = idv & (_WIN - 1)
            colv = jnp.full((_L,), col, jnp.int32)
            slot = j & 127
            chunks = []
            ss = jnp.zeros((_L,), jnp.float32)
            for k in range(EMBED // _L):
                idx_c = lax.iota(jnp.int32, _L) + k * _L
                r = plsc.load_gather(wref, [idx_c, colv])
                chunks.append(r)
                ss = ss + r * r
            y = _rsqrt_nr(jnp.maximum(_lane_sum(ss), jnp.float32(1e-24)))
            for k in range(EMBED // _L):
                ring[slot, pl.ds(k * _L, _L)] = chunks[k] * y
            pltpu.make_async_copy(
                ring.at[slot], out_hbm.at[pl.ds(pos * EMBED, EMBED)], wsem
            ).start()

        def process_win(l, b):
            c0 = starts[l]
            c1 = starts[l + 1]
            wref = buf.at[b]
            def jbody(j, c):
                process_id(j, wref)
                return c
            lax.fori_loop(c0, c1, jbody, 0)

        start_win(0, 0, sem_a)
        start_win(1, 1, sem_b)

        def lbody(k, c):
            l0 = k * 2
            wait_win(0, sem_a)
            process_win(l0, 0)
            @pl.when(k < _WPT // 2 - 1)
            def _():
                start_win(l0 + 2, 0, sem_a)
            wait_win(1, sem_b)
            process_win(l0 + 1, 1)
            @pl.when(k < _WPT // 2 - 1)
            def _():
                start_win(l0 + 3, 1, sem_b)
            return c
        lax.fori_loop(0, _WPT // 2, lbody, 0)

        @pl.when(wid <= 3)
        def _():
            pltpu.sync_copy(
                tt_hbm.at[:, pl.ds((_WPT * _NW + wid) * _WIN, _WIN)],
                buf.at[0])
            process_win(_WPT, 0)

        @pl.when(wid == 4)
        def _():
            pltpu.sync_copy(tt_hbm.at[:, pl.ds(_TAIL0, EMBED)], tailbuf)
            c0 = starts[_WPT]
            c1 = starts[_WPT + 1]
            def jbody(j, c):
                process_id(j, tailbuf)
                return c
            lax.fori_loop(c0, c1, jbody, 0)

        return cnt

    ucnt = run_table(uall, usel_id, usel_pos, uord_id, uord_pos, ustarts,
                     utt_hbm, uout_hbm, ubuf, uring, usem0, usem1, uwsem)
    icnt = run_table(iall, isel_id, isel_pos, iord_id, iord_pos, istarts,
                     itt_hbm, iout_hbm, ibuf, iring, isem0, isem1, iwsem)

    def drain(n, out_hbm, wsem):
        def dbody(j, c):
            pltpu.make_async_copy(
                out_hbm.at[pl.ds(0, EMBED)], drain_dst, wsem
            ).wait()
            return c
        lax.fori_loop(0, jnp.minimum(n, 128), dbody, 0)

    drain(ucnt, uout_hbm, uwsem)
    drain(icnt, iout_hbm, iwsem)


def kernel(user_ids, user_features, item_ids, item_features,
           user_table, item_table):
    del user_features, item_features
    uids = user_ids.astype(jnp.int32)
    iids = item_ids.astype(jnp.int32)
    uf, itf = _emb_kernel(uids, iids, user_table.T, item_table.T)
    return uf.reshape(BATCH, EMBED), itf.reshape(BATCH, EMBED)


# speedup vs baseline: 3.8546x; 1.4629x over previous
import functools

import jax
import jax.numpy as jnp
from jax import lax
from jax.experimental import pallas as pl
from jax.experimental.pallas import tpu as pltpu
from jax.experimental.pallas import tpu_sc as plsc

BATCH = 16384
EMBED = 64
NROWS = 1000000

_info = plsc.get_sparse_core_info()
_NC, _NS, _L = _info.num_cores, _info.num_subcores, _info.num_lanes
_NW = _NC * _NS
_WIN = 128
_NWIN = NROWS // _WIN
_WPT = _NWIN // _NW
_TAIL0 = _NWIN * _WIN
_CAP = 2048
_LCAP = _CAP + 16


def _rsqrt_nr(sv):
    bits = lax.bitcast_convert_type(sv, jnp.int32)
    y = lax.bitcast_convert_type(jnp.int32(0x5F3759DF) - (bits >> 1),
                                 jnp.float32)
    for _ in range(3):
        y = y * (1.5 - 0.5 * sv * y * y)
    return y


def _shuffle(x, idx):
    dnums = lax.GatherDimensionNumbers(
        offset_dims=(), collapsed_slice_dims=(0,), start_index_map=(0,))
    return lax.gather(x, idx[:, None], dnums, slice_sizes=(1,),
                      mode=lax.GatherScatterMode.PROMISE_IN_BOUNDS)


def _lane_sum(x):
    lanes = lax.iota(jnp.int32, _L)
    for sh in (8, 4, 2, 1):
        x = x + _shuffle(x, lanes ^ sh)
    return x


def _prefix_sum(x):
    lanes = lax.iota(jnp.int32, _L)
    for k, sh in enumerate((1, 2, 4, 8)):
        g = jnp.minimum(lanes, sh) >> k
        x = x + _shuffle(x, jnp.maximum(lanes - sh, 0)) * g
    return x


_mesh = plsc.VectorSubcoreMesh(core_axis_name="c", subcore_axis_name="s")


@functools.partial(
    pl.kernel,
    out_type=(jax.ShapeDtypeStruct((BATCH * EMBED,), jnp.float32),
              jax.ShapeDtypeStruct((BATCH * EMBED,), jnp.float32)),
    mesh=_mesh,
    compiler_params=pltpu.CompilerParams(needs_layout_passes=False),
    scratch_types=[
        pltpu.VMEM((BATCH,), jnp.int32),
        pltpu.VMEM((_LCAP,), jnp.int32),
        pltpu.VMEM((_LCAP,), jnp.int32),
        pltpu.VMEM((_LCAP,), jnp.int32),
        pltpu.VMEM((_LCAP,), jnp.int32),
        pltpu.VMEM((_LCAP,), jnp.int32),
        pltpu.VMEM((_LCAP,), jnp.int32),
        pltpu.VMEM((272,), jnp.int32),
        pltpu.SMEM((272,), jnp.int32),
        pltpu.SMEM((272,), jnp.int32),
        pltpu.SMEM((272,), jnp.int32),
        pltpu.SMEM((272,), jnp.int32),
        pltpu.SMEM((272,), jnp.int32),
        pltpu.VMEM((2, EMBED, _WIN), jnp.float32),
        pltpu.VMEM((2, EMBED, _WIN), jnp.float32),
        pltpu.VMEM((EMBED, EMBED), jnp.float32),
        pltpu.VMEM((128, EMBED), jnp.float32),
        pltpu.VMEM((128, EMBED), jnp.float32),
        pltpu.VMEM((EMBED,), jnp.float32),
        pltpu.SemaphoreType.DMA,
        pltpu.SemaphoreType.DMA,
        pltpu.SemaphoreType.DMA,
        pltpu.SemaphoreType.DMA,
        pltpu.SemaphoreType.DMA,
        pltpu.SemaphoreType.DMA,
    ],
)
def _emb_kernel(uids_hbm, iids_hbm, utt_hbm, itt_hbm,
                uout_hbm, iout_hbm,
                all_stage, sel_id, sel_pos,
                uord_id, uord_pos, iord_id, iord_pos,
                hist, ustarts, istarts, cursor, uwinlist, iwinlist,
                ubuf, ibuf, tailbuf, uring, iring, drain_dst,
                usem0, usem1, isem0, isem1, uwsem, iwsem):
    wid = lax.axis_index("s") * _NC + lax.axis_index("c")

    def sload(ref, j):
        return ref[pl.ds(j, _L)][0]

    def sstore(ref, j, val):
        plsc.store_scatter(ref, [jnp.full((_L,), j, jnp.int32)],
                           jnp.full((_L,), val, jnp.int32))

    def start_win(tt_hbm, l, bref, sem):
        win = l * _NW + wid
        pltpu.make_async_copy(
            tt_hbm.at[:, pl.ds(win * _WIN, _WIN)], bref, sem).start()

    def wait_win(tt_hbm, bref, sem):
        pltpu.make_async_copy(
            tt_hbm.at[:, pl.ds(0, _WIN)], bref, sem).wait()

    start_win(utt_hbm, 0, ubuf.at[0], usem0)
    start_win(itt_hbm, 0, ibuf.at[0], isem0)
    start_win(utt_hbm, 1, ubuf.at[1], usem1)
    start_win(itt_hbm, 1, ibuf.at[1], isem1)

    trash = jnp.full((_L,), _CAP + 8, jnp.int32)
    lanes15 = jnp.full((_L,), _L - 1, jnp.int32)
    ones = jnp.ones((_L,), jnp.int32)

    def preprocess(ids_hbm, ord_id, ord_pos, starts, winlist):
        pltpu.sync_copy(ids_hbm, all_stage)

        def sbody(v, cnt_vec):
            vec = all_stage[pl.ds(v * _L, _L)]
            diff = ((vec >> 7) & (_NW - 1)) ^ wid
            mi = ones - jnp.minimum(diff, ones)
            posv = lax.iota(jnp.int32, _L) + v * _L
            inc = _prefix_sum(mi)
            dest = (cnt_vec + inc - mi) * mi + trash * (ones - mi)
            plsc.store_scatter(sel_id, [dest], vec)
            plsc.store_scatter(sel_pos, [dest], posv)
            return jnp.minimum(cnt_vec + _shuffle(inc, lanes15), _CAP)
        cnt_vec = lax.fori_loop(0, BATCH // _L, sbody,
                                jnp.zeros((_L,), jnp.int32), unroll=4)
        cnt = cnt_vec[0]

        plsc.store_scatter(
            sel_id, [cnt_vec + lax.iota(jnp.int32, _L)],
            jnp.full((_L,), 260 << 12, jnp.int32))

        zeros16 = jnp.zeros((_L,), jnp.int32)
        def zbody(v, c):
            hist[pl.ds(v * _L, _L)] = zeros16
            return c
        lax.fori_loop(0, 272 // _L, zbody, 0)

        def hbody(v, c):
            lw = sel_id[pl.ds(v * _L, _L)] >> 12
            plsc.addupdate_scatter(hist, [lw], ones)
            return c
        lax.fori_loop(0, (cnt + _L - 1) // _L, hbody, 0)

        def pbody(l, carry):
            s, nw = carry
            starts[l] = s
            cursor[l] = s
            h = sload(hist, l)
            @pl.when(jnp.logical_and(h > 0, l < _WPT))
            def _():
                winlist[nw] = l
            nw = nw + jnp.where(jnp.logical_and(h > 0, l < _WPT), 1, 0)
            return (s + h, nw)
        _, nwin = lax.fori_loop(0, 246, pbody, (0, 0))

        def scbody(j, c):
            idv = sload(sel_id, j)
            lw = idv >> 12
            o = cursor[lw]
            cursor[lw] = o + 1
            sstore(ord_id, o, idv)
            sstore(ord_pos, o, sload(sel_pos, j))
            return c
        lax.fori_loop(0, cnt, scbody, 0)
        return cnt, nwin

    ucnt, unwin = preprocess(uids_hbm, uord_id, uord_pos, ustarts, uwinlist)
    icnt, inwin = preprocess(iids_hbm, iord_id, iord_pos, istarts, iwinlist)

    def make_proc(ord_id, ord_pos, ring, out_hbm, wsem, starts):
        def process_id(j, wref):
            @pl.when(j >= 128)
            def _():
                pltpu.make_async_copy(
                    out_hbm.at[pl.ds(0, EMBED)], drain_dst, wsem
                ).wait()
            idv = sload(ord_id, j)
            pos = sload(ord_pos, j)
            col = idv & (_WIN - 1)
            colv = jnp.full((_L,), col, jnp.int32)
            slot = j & 127
            chunks = []
            ss = jnp.zeros((_L,), jnp.float32)
            for k in range(EMBED // _L):
                idx_c = lax.iota(jnp.int32, _L) + k * _L
                r = plsc.load_gather(wref, [idx_c, colv])
                chunks.append(r)
                ss = ss + r * r
            y = _rsqrt_nr(jnp.maximum(_lane_sum(ss), jnp.float32(1e-24)))
            for k in range(EMBED // _L):
                ring[slot, pl.ds(k * _L, _L)] = chunks[k] * y
            pltpu.make_async_copy(
                ring.at[slot], out_hbm.at[pl.ds(pos * EMBED, EMBED)], wsem
            ).start()

        def process_win(l, wref):
            c0 = starts[l]
            c1 = starts[l + 1]
            def jbody(j, c):
                process_id(j, wref)
                return c
            lax.fori_loop(c0, c1, jbody, 0)

        return process_id, process_win

    uproc_id, uproc_win = make_proc(uord_id, uord_pos, uring, uout_hbm,
                                    uwsem, ustarts)
    iproc_id, iproc_win = make_proc(iord_id, iord_pos, iring, iout_hbm,
                                    iwsem, istarts)

    wait_win(utt_hbm, ubuf.at[0], usem0)
    wait_win(utt_hbm, ubuf.at[1], usem1)
    wait_win(itt_hbm, ibuf.at[0], isem0)
    wait_win(itt_hbm, ibuf.at[1], isem1)

    def prime2(tt_hbm, buf, sems, winlist, nwin):
        for m in range(2):
            @pl.when(m < nwin)
            def _():
                start_win(tt_hbm, winlist[m], buf.at[m], sems[m])
    prime2(utt_hbm, ubuf, usems2 := (usem0, usem1), uwinlist, unwin)
    prime2(itt_hbm, ibuf, isems2 := (isem0, isem1), iwinlist, inwin)

    usems = (usem0, usem1)
    isems = (isem0, isem1)

    def step(tt_hbm, buf, sems, winlist, nwin, proc_win, m, half):
        @pl.when(m < nwin)
        def _():
            wait_win(tt_hbm, buf.at[half], sems[half])
            proc_win(winlist[m], buf.at[half])
            @pl.when(m + 2 < nwin)
            def _():
                start_win(tt_hbm, winlist[m + 2], buf.at[half],
                          sems[half])

    def lbody(k, c):
        for half in (0, 1):
            m = k * 2 + half
            step(utt_hbm, ubuf, usems, uwinlist, unwin, uproc_win,
                 m, half)
            step(itt_hbm, ibuf, isems, iwinlist, inwin, iproc_win,
                 m, half)
        return c
    lax.fori_loop(0, (jnp.maximum(unwin, inwin) + 1) // 2, lbody, 0)

    @pl.when(wid <= 3)
    def _():
        off = (_WPT * _NW + wid) * _WIN
        pltpu.sync_copy(utt_hbm.at[:, pl.ds(off, _WIN)], ubuf.at[0])
        uproc_win(_WPT, ubuf.at[0])
        pltpu.sync_copy(itt_hbm.at[:, pl.ds(off, _WIN)], ibuf.at[0])
        iproc_win(_WPT, ibuf.at[0])

    @pl.when(wid == 4)
    def _():
        pltpu.sync_copy(utt_hbm.at[:, pl.ds(_TAIL0, EMBED)], tailbuf)
        def ujbody(j, c):
            uproc_id(j, tailbuf)
            return c
        lax.fori_loop(ustarts[_WPT], ustarts[_WPT + 1], ujbody, 0)
        pltpu.sync_copy(itt_hbm.at[:, pl.ds(_TAIL0, EMBED)], tailbuf)
        def ijbody(j, c):
            iproc_id(j, tailbuf)
            return c
        lax.fori_loop(istarts[_WPT], istarts[_WPT + 1], ijbody, 0)

    def drain(n, out_hbm, wsem):
        def dbody(j, c):
            pltpu.make_async_copy(
                out_hbm.at[pl.ds(0, EMBED)], drain_dst, wsem
            ).wait()
            return c
        lax.fori_loop(0, jnp.minimum(n, 128), dbody, 0)

    drain(ucnt, uout_hbm, uwsem)
    drain(icnt, iout_hbm, iwsem)


def kernel(user_ids, user_features, item_ids, item_features,
           user_table, item_table):
    del user_features, item_features
    uids = user_ids.astype(jnp.int32)
    iids = item_ids.astype(jnp.int32)
    uf, itf = _emb_kernel(uids, iids, user_table.T, item_table.T)
    return uf.reshape(BATCH, EMBED), itf.reshape(BATCH, EMBED)
